# baseline (speedup 1.0000x reference)
import functools
import jax
import jax.numpy as jnp
from jax import lax
from jax.experimental import pallas as pl
from jax.experimental.pallas import tpu as pltpu
from jax.experimental.pallas import tpu_sc as plsc

F = 26
B = 4096
V = 100000
D = 64
NLANE = 16
NC = 2
NS = 16
NW = NC * NS
NB = 128
NBLK = B // NB
C = 512
CJ = C // 128
KD = D // NLANE
GROUPS = (7, 7, 6, 6)


def _sload(ref, i):
    return ref[pl.ds(i, NLANE)][0]


def _make_body(f0, fg):
    ipw = fg * NBLK // NW

    def _sc_body(vals_hbm, offs_hbm, tab_hbm, out_hbm,
                 offs_v, vals_v, idx_v, rows_v, out_v, sems):
        w = lax.axis_index("s") * NC + lax.axis_index("c")

        def item_loop(i, _):
            item = w * ipw + i
            fi = item // NBLK
            bag0 = (item % NBLK) * NB
            goff = (f0 + fi) * B + bag0

            pltpu.sync_copy(offs_hbm.at[pl.ds(goff, NB + 32)], offs_v)

            def zero_loop(z, _):
                for k in range(KD):
                    out_v[z, pl.ds(k * NLANE, NLANE)] = jnp.zeros(
                        (NLANE,), jnp.float32)
                return 0

            lax.fori_loop(0, NB, zero_loop, 0)

            p_start = _sload(offs_v, 0)
            p_end = _sload(offs_v, NB)
            p8 = (p_start // 8) * 8
            nsub = (p_end - p8 + (C - 1)) // C
            fV = fi * V

            def fire(fsub, par):
                base = p8 + fsub * C
                pltpu.sync_copy(vals_hbm.at[pl.ds(base, C)], vals_v)
                for j in range(CJ):
                    for k in range(128 // NLANE):
                        idx_v[par, j, pl.ds(k * NLANE, NLANE)] = (
                            vals_v[pl.ds(j * 128 + k * NLANE, NLANE)] + fV)
                for j in range(CJ):
                    pltpu.async_copy(
                        tab_hbm.at[idx_v.at[par, j]],
                        rows_v.at[par, pl.ds(j * 128, 128)], sems.at[par])

            @pl.when(nsub > 0)
            def _():
                fire(jnp.int32(0), jnp.int32(0))

            def sub_loop(sub, carry):
                s, a0, a1, a2, a3 = carry
                par = lax.rem(sub, 2)

                @pl.when(sub + 1 < nsub)
                def _():
                    fire(sub + 1, 1 - par)

                pltpu.make_async_copy(
                    tab_hbm.at[pl.ds(0, C)], rows_v.at[par],
                    sems.at[par]).wait()

                base = p8 + sub * C
                lim = base + C

                def bs_step(_, lh):
                    blo, bhi = lh
                    mid = (blo + bhi) // 2
                    pred = _sload(offs_v, mid) < lim
                    blo = jnp.where(pred, mid + 1, blo)
                    bhi = jnp.where(pred, bhi, mid)
                    return (blo, bhi)

                s_end, _ = lax.fori_loop(0, 8, bs_step,
                                         (s, jnp.int32(NB)))

                def bag_body(sb, a):
                    a0, a1, a2, a3 = a
                    o_pair = offs_v[pl.ds(sb, NLANE)]
                    o_lo = o_pair[0]
                    o_hi = o_pair[1]
                    lo = jnp.maximum(o_lo, base)
                    hi = jnp.minimum(o_hi, lim)

                    def row_body(r, aa):
                        lr = r - base
                        return (aa[0] + rows_v[par, lr, pl.ds(0, NLANE)],
                                aa[1] + rows_v[par, lr,
                                               pl.ds(NLANE, NLANE)],
                                aa[2] + rows_v[par, lr,
                                               pl.ds(2 * NLANE, NLANE)],
                                aa[3] + rows_v[par, lr,
                                               pl.ds(3 * NLANE, NLANE)])

                    a0, a1, a2, a3 = lax.fori_loop(lo, hi, row_body,
                                                   (a0, a1, a2, a3))
                    done = o_hi <= lim
                    srow = jnp.where(done, sb, jnp.int32(NB))
                    out_v[srow, pl.ds(0, NLANE)] = a0
                    out_v[srow, pl.ds(NLANE, NLANE)] = a1
                    out_v[srow, pl.ds(2 * NLANE, NLANE)] = a2
                    out_v[srow, pl.ds(3 * NLANE, NLANE)] = a3
                    zero = jnp.zeros((NLANE,), jnp.float32)
                    a0 = jnp.where(done, zero, a0)
                    a1 = jnp.where(done, zero, a1)
                    a2 = jnp.where(done, zero, a2)
                    a3 = jnp.where(done, zero, a3)
                    return (a0, a1, a2, a3)

                a0, a1, a2, a3 = lax.fori_loop(s, s_end, bag_body,
                                               (a0, a1, a2, a3))
                last = jnp.maximum(s_end - 1, s)
                incomplete = _sload(offs_v, last + 1) > lim
                s = jnp.where(s_end > s,
                              s_end - incomplete.astype(jnp.int32), s)
                return (s, a0, a1, a2, a3)

            zero = jnp.zeros((NLANE,), jnp.float32)
            lax.fori_loop(0, nsub, sub_loop,
                          (jnp.int32(0), zero, zero, zero, zero))

            pltpu.sync_copy(out_v.at[pl.ds(0, NB)],
                            out_hbm.at[pl.ds(bag0, NB), fi])
            return 0

        lax.fori_loop(0, ipw, item_loop, 0)

    return _sc_body


def _group_call(f0, fg, vals_pad, offs_pad, tab_flat):
    mesh = plsc.VectorSubcoreMesh(core_axis_name="c", subcore_axis_name="s")
    return pl.kernel(
        _make_body(f0, fg),
        out_type=jax.ShapeDtypeStruct((B, fg, D), jnp.float32),
        mesh=mesh,
        compiler_params=pltpu.CompilerParams(use_tc_tiling_on_sc=False),
        name=f"ebc_g{f0}",
        scratch_types=[
            pltpu.VMEM((NB + 32,), jnp.int32),
            pltpu.VMEM((C,), jnp.int32),
            pltpu.VMEM((2, CJ, 128), jnp.int32),
            pltpu.VMEM((2, C, D), jnp.float32),
            pltpu.VMEM((NB + 1, D), jnp.float32),
            pltpu.SemaphoreType.DMA((2,)),
        ],
    )(vals_pad, offs_pad, tab_flat)


@jax.jit
def _ebc(values, offsets, tables):
    total = values.shape[0]
    vals_pad = jnp.concatenate(
        [values, jnp.zeros((C + 8,), jnp.int32)])
    offs_pad = jnp.concatenate(
        [offsets, jnp.full((NB + 64,), jnp.int32(total))])
    outs = []
    f0 = 0
    for fg in GROUPS:
        tab_g = lax.slice_in_dim(tables, f0, f0 + fg, axis=0)
        tab_flat = tab_g.reshape(fg * V, D)
        outs.append(_group_call(f0, fg, vals_pad, offs_pad, tab_flat))
        f0 += fg
    return jnp.concatenate(outs, axis=1).reshape(B, F * D)


def kernel(values, offsets, tables):
    return _ebc(values, offsets, tables)


# speedup vs baseline: 1.1605x; 1.1605x over previous
import jax
import jax.numpy as jnp
from jax import lax
from jax.experimental import pallas as pl
from jax.experimental.pallas import tpu as pltpu
from jax.experimental.pallas import tpu_sc as plsc

F = 26
B = 4096
V = 100000
D = 64
NLANE = 16
NC = 2
NS = 16
NW = NC * NS
NB = 256
IPF = B // NB
NITEMS = F * IPF
IPW = NITEMS // NW
C = 512
CJ = C // 128
KD = D // NLANE


def _sload(ref, i):
    return ref[pl.ds(i, NLANE)][0]


def _sc_body(vals_hbm, offs_hbm, tab_hbm, out_hbm,
             offs_v, idx_v, rows_v, out_v, sems):
    w = lax.axis_index("s") * NC + lax.axis_index("c")

    def item_loop(i, _):
        item = w * IPW + i
        f = item // IPF
        bag0 = (item % IPF) * NB
        goff = f * B + bag0

        pltpu.sync_copy(offs_hbm.at[pl.ds(goff, NB + 32)], offs_v)

        def zero_loop(z, _):
            for k in range(KD):
                out_v[z, pl.ds(k * NLANE, NLANE)] = jnp.zeros((NLANE,),
                                                              jnp.float32)
            return 0

        lax.fori_loop(0, NB, zero_loop, 0)

        p_start = _sload(offs_v, 0)
        p_end = _sload(offs_v, NB)
        p8 = (p_start // 8) * 8
        nsub = (p_end - p8 + (C - 1)) // C

        def fire(fsub, par):
            base = p8 + fsub * C
            for j in range(CJ):
                pltpu.sync_copy(
                    vals_hbm.at[pl.ds(base + j * 128, 128)],
                    idx_v.at[par, j])
            for j in range(CJ):
                pltpu.async_copy(
                    tab_hbm.at[f].at[idx_v.at[par, j]],
                    rows_v.at[par, pl.ds(j * 128, 128)], sems.at[par])

        @pl.when(nsub > 0)
        def _():
            fire(jnp.int32(0), jnp.int32(0))

        def sub_loop(sub, carry):
            s, a0, a1, a2, a3 = carry
            par = lax.rem(sub, 2)

            @pl.when(sub + 1 < nsub)
            def _():
                fire(sub + 1, 1 - par)

            pltpu.make_async_copy(
                tab_hbm.at[0].at[pl.ds(0, C)], rows_v.at[par],
                sems.at[par]).wait()

            base = p8 + sub * C
            lim = base + C

            def bs_step(_, lh):
                blo, bhi = lh
                mid = (blo + bhi) // 2
                pred = _sload(offs_v, mid) < lim
                blo = jnp.where(pred, mid + 1, blo)
                bhi = jnp.where(pred, bhi, mid)
                return (blo, bhi)

            s_end, _ = lax.fori_loop(0, 9, bs_step, (s, jnp.int32(NB)))

            def bag_body(sb, a):
                a0, a1, a2, a3 = a
                o_pair = offs_v[pl.ds(sb, NLANE)]
                o_lo = o_pair[0]
                o_hi = o_pair[1]
                lo = jnp.maximum(o_lo, base)
                hi = jnp.minimum(o_hi, lim)

                def row_body(r, aa):
                    lr = r - base
                    return (aa[0] + rows_v[par, lr, pl.ds(0, NLANE)],
                            aa[1] + rows_v[par, lr, pl.ds(NLANE, NLANE)],
                            aa[2] + rows_v[par, lr, pl.ds(2 * NLANE, NLANE)],
                            aa[3] + rows_v[par, lr, pl.ds(3 * NLANE, NLANE)])

                a0, a1, a2, a3 = lax.fori_loop(lo, hi, row_body,
                                               (a0, a1, a2, a3))
                done = o_hi <= lim
                srow = jnp.where(done, sb, jnp.int32(NB))
                out_v[srow, pl.ds(0, NLANE)] = a0
                out_v[srow, pl.ds(NLANE, NLANE)] = a1
                out_v[srow, pl.ds(2 * NLANE, NLANE)] = a2
                out_v[srow, pl.ds(3 * NLANE, NLANE)] = a3
                zero = jnp.zeros((NLANE,), jnp.float32)
                a0 = jnp.where(done, zero, a0)
                a1 = jnp.where(done, zero, a1)
                a2 = jnp.where(done, zero, a2)
                a3 = jnp.where(done, zero, a3)
                return (a0, a1, a2, a3)

            a0, a1, a2, a3 = lax.fori_loop(s, s_end, bag_body,
                                           (a0, a1, a2, a3))
            last = jnp.maximum(s_end - 1, s)
            incomplete = _sload(offs_v, last + 1) > lim
            s = jnp.where(s_end > s,
                          s_end - incomplete.astype(jnp.int32), s)
            return (s, a0, a1, a2, a3)

        zero = jnp.zeros((NLANE,), jnp.float32)
        lax.fori_loop(0, nsub, sub_loop,
                      (jnp.int32(0), zero, zero, zero, zero))

        pltpu.sync_copy(out_v.at[pl.ds(0, NB)],
                        out_hbm.at[pl.ds(bag0, NB), f])
        return 0

    lax.fori_loop(0, IPW, item_loop, 0)


@jax.jit
def _ebc_sc(vals_pad, offs_pad, tab3):
    mesh = plsc.VectorSubcoreMesh(core_axis_name="c", subcore_axis_name="s")
    return pl.kernel(
        _sc_body,
        out_type=jax.ShapeDtypeStruct((B, F, D), jnp.float32),
        mesh=mesh,
        compiler_params=pltpu.CompilerParams(use_tc_tiling_on_sc=False),
        scratch_types=[
            pltpu.VMEM((NB + 32,), jnp.int32),
            pltpu.VMEM((2, CJ, 128), jnp.int32),
            pltpu.VMEM((2, C, D), jnp.float32),
            pltpu.VMEM((NB + 1, D), jnp.float32),
            pltpu.SemaphoreType.DMA((2,)),
        ],
    )(vals_pad, offs_pad, tab3)


def kernel(values, offsets, tables):
    total = values.shape[0]
    vals_pad = jnp.concatenate(
        [values, jnp.zeros((C + 8,), jnp.int32)])
    offs_pad = jnp.concatenate(
        [offsets, jnp.full((NB + 64,), jnp.int32(total))])
    pooled = _ebc_sc(vals_pad, offs_pad, tables)
    return pooled.reshape(B, F * D)


# speedup vs baseline: 1.2404x; 1.0689x over previous
import jax
import jax.numpy as jnp
from jax import lax
from jax.experimental import pallas as pl
from jax.experimental.pallas import tpu as pltpu
from jax.experimental.pallas import tpu_sc as plsc

F = 26
B = 4096
V = 100000
D = 64
NLANE = 16
NC = 2
NS = 16
NW = NC * NS
NB = 256
IPF = B // NB
NITEMS = F * IPF
IPW = NITEMS // NW
C = 512
CJ = C // 128
KD = D // NLANE


def _sload(ref, i):
    return ref[pl.ds(i, NLANE)][0]


def _sc_body(vals_hbm, offs_hbm, tab_hbm, out_hbm,
             offs_v, idx_v, rows_v, out_v, sems):
    w = lax.axis_index("s") * NC + lax.axis_index("c")

    def item_loop(i, _):
        item = w * IPW + i
        f = item // IPF
        bag0 = (item % IPF) * NB
        goff = f * B + bag0

        pltpu.sync_copy(offs_hbm.at[pl.ds(goff, NB + 32)], offs_v)

        def zero_loop(z, _):
            for k in range(KD):
                out_v[z, pl.ds(k * NLANE, NLANE)] = jnp.zeros((NLANE,),
                                                              jnp.float32)
            return 0

        lax.fori_loop(0, NB, zero_loop, 0)

        p_start = _sload(offs_v, 0)
        p_end = _sload(offs_v, NB)
        p8 = (p_start // 8) * 8
        nsub = (p_end - p8 + (C - 1)) // C

        def fire(fsub, par):
            base = p8 + fsub * C
            pltpu.sync_copy(vals_hbm.at[pl.ds(base, C)],
                            idx_v.at[pl.ds(par * C, C)])
            for j in range(CJ):
                pltpu.async_copy(
                    tab_hbm.at[f].at[idx_v.at[pl.ds(par * C + j * 128,
                                                    128)]],
                    rows_v.at[par, pl.ds(j * 128, 128)], sems.at[par])

        @pl.when(nsub > 0)
        def _():
            fire(jnp.int32(0), jnp.int32(0))

        def sub_loop(sub, carry):
            s, a0, a1, a2, a3 = carry
            par = lax.rem(sub, 2)

            @pl.when(sub + 1 < nsub)
            def _():
                fire(sub + 1, 1 - par)

            pltpu.make_async_copy(
                tab_hbm.at[0].at[pl.ds(0, C)], rows_v.at[par],
                sems.at[par]).wait()

            base = p8 + sub * C
            lim = base + C

            def bs_step(_, lh):
                blo, bhi = lh
                mid = (blo + bhi) // 2
                pred = _sload(offs_v, mid) < lim
                blo = jnp.where(pred, mid + 1, blo)
                bhi = jnp.where(pred, bhi, mid)
                return (blo, bhi)

            s_end, _ = lax.fori_loop(0, 9, bs_step, (s, jnp.int32(NB)))

            def bag_body(sb, a):
                a0, a1, a2, a3 = a
                o_pair = offs_v[pl.ds(sb, NLANE)]
                o_lo = o_pair[0]
                o_hi = o_pair[1]
                lo = jnp.maximum(o_lo, base)
                hi = jnp.minimum(o_hi, lim)

                def row_body(r, aa):
                    lr = r - base
                    return (aa[0] + rows_v[par, lr, pl.ds(0, NLANE)],
                            aa[1] + rows_v[par, lr, pl.ds(NLANE, NLANE)],
                            aa[2] + rows_v[par, lr, pl.ds(2 * NLANE, NLANE)],
                            aa[3] + rows_v[par, lr, pl.ds(3 * NLANE, NLANE)])

                a0, a1, a2, a3 = lax.fori_loop(lo, hi, row_body,
                                               (a0, a1, a2, a3))
                done = o_hi <= lim
                srow = jnp.where(done, sb, jnp.int32(NB))
                out_v[srow, pl.ds(0, NLANE)] = a0
                out_v[srow, pl.ds(NLANE, NLANE)] = a1
                out_v[srow, pl.ds(2 * NLANE, NLANE)] = a2
                out_v[srow, pl.ds(3 * NLANE, NLANE)] = a3
                zero = jnp.zeros((NLANE,), jnp.float32)
                a0 = jnp.where(done, zero, a0)
                a1 = jnp.where(done, zero, a1)
                a2 = jnp.where(done, zero, a2)
                a3 = jnp.where(done, zero, a3)
                return (a0, a1, a2, a3)

            a0, a1, a2, a3 = lax.fori_loop(s, s_end, bag_body,
                                           (a0, a1, a2, a3))
            last = jnp.maximum(s_end - 1, s)
            incomplete = _sload(offs_v, last + 1) > lim
            s = jnp.where(s_end > s,
                          s_end - incomplete.astype(jnp.int32), s)
            return (s, a0, a1, a2, a3)

        zero = jnp.zeros((NLANE,), jnp.float32)
        lax.fori_loop(0, nsub, sub_loop,
                      (jnp.int32(0), zero, zero, zero, zero))

        pltpu.sync_copy(out_v.at[pl.ds(0, NB)],
                        out_hbm.at[pl.ds(bag0, NB), f])
        return 0

    lax.fori_loop(0, IPW, item_loop, 0)


@jax.jit
def _ebc_sc(vals_pad, offs_pad, tab3):
    mesh = plsc.VectorSubcoreMesh(core_axis_name="c", subcore_axis_name="s")
    return pl.kernel(
        _sc_body,
        out_type=jax.ShapeDtypeStruct((B, F, D), jnp.float32),
        mesh=mesh,
        compiler_params=pltpu.CompilerParams(use_tc_tiling_on_sc=False),
        scratch_types=[
            pltpu.VMEM((NB + 32,), jnp.int32),
            pltpu.VMEM((2 * C,), jnp.int32),
            pltpu.VMEM((2, C, D), jnp.float32),
            pltpu.VMEM((NB + 1, D), jnp.float32),
            pltpu.SemaphoreType.DMA((2,)),
        ],
    )(vals_pad, offs_pad, tab3)


def kernel(values, offsets, tables):
    total = values.shape[0]
    vals_pad = jnp.concatenate(
        [values, jnp.zeros((C + 8,), jnp.int32)])
    offs_pad = jnp.concatenate(
        [offsets, jnp.full((NB + 64,), jnp.int32(total))])
    pooled = _ebc_sc(vals_pad, offs_pad, tables)
    return pooled.reshape(B, F * D)


# speedup vs baseline: 1.3017x; 1.0494x over previous
import jax
import jax.numpy as jnp
from jax import lax
from jax.experimental import pallas as pl
from jax.experimental.pallas import tpu as pltpu
from jax.experimental.pallas import tpu_sc as plsc

F = 26
B = 4096
V = 100000
D = 64
NLANE = 16
NC = 2
NS = 16
NW = NC * NS
NB = 256
IPF = B // NB
NITEMS = F * IPF
IPW = NITEMS // NW
C = 512
CJ = C // 128
KD = D // NLANE


def _sload(ref, i):
    return ref[pl.ds(i, NLANE)][0]


def _sc_body(vals_hbm, offs_hbm, tab_hbm, out_hbm,
             offs_v, idx_v, rows_v, out_v, sems):
    w = lax.axis_index("s") * NC + lax.axis_index("c")

    def item_loop(i, _):
        item = w * IPW + i
        f = item // IPF
        bag0 = (item % IPF) * NB
        goff = f * B + bag0

        pltpu.sync_copy(offs_hbm.at[pl.ds(goff, NB + 32)], offs_v)

        def zero_loop(z, _):
            for k in range(KD):
                out_v[z, pl.ds(k * NLANE, NLANE)] = jnp.zeros((NLANE,),
                                                              jnp.float32)
            return 0

        lax.fori_loop(0, NB, zero_loop, 0)

        p_start = _sload(offs_v, 0)
        p_end = _sload(offs_v, NB)
        p8 = (p_start // 8) * 8
        nsub = (p_end - p8 + (C - 1)) // C

        def fire(fsub, par):
            base = p8 + fsub * C
            pltpu.sync_copy(vals_hbm.at[pl.ds(base, C)],
                            idx_v.at[pl.ds(par * C, C)])
            for j in range(CJ):
                pltpu.async_copy(
                    tab_hbm.at[f].at[idx_v.at[pl.ds(par * C + j * 128,
                                                    128)]],
                    rows_v.at[par, pl.ds(j * 128, 128)], sems.at[par])

        @pl.when(nsub > 0)
        def _():
            fire(jnp.int32(0), jnp.int32(0))

        def sub_loop(sub, carry):
            s, a0, a1, a2, a3 = carry
            par = lax.rem(sub, 2)

            @pl.when(sub + 1 < nsub)
            def _():
                fire(sub + 1, 1 - par)

            pltpu.make_async_copy(
                tab_hbm.at[0].at[pl.ds(0, C)], rows_v.at[par],
                sems.at[par]).wait()

            base = p8 + sub * C
            lim = base + C

            def bs_step(_, lh):
                blo, bhi = lh
                mid = (blo + bhi) // 2
                pred = _sload(offs_v, mid) < lim
                blo = jnp.where(pred, mid + 1, blo)
                bhi = jnp.where(pred, bhi, mid)
                return (blo, bhi)

            s_end, _ = lax.fori_loop(0, 9, bs_step, (s, jnp.int32(NB)))

            def bag_body(sb, a):
                a0, a1, a2, a3 = a
                o_pair = offs_v[pl.ds(sb, NLANE)]
                o_lo = o_pair[0]
                o_hi = o_pair[1]
                lo = jnp.maximum(o_lo, base)
                hi = jnp.minimum(o_hi, lim)

                lob = lo - base
                npair = (hi - lo) // 2

                def row2_body(t, aa):
                    lr = lob + 2 * t
                    x0 = (rows_v[par, lr, pl.ds(0, NLANE)]
                          + rows_v[par, lr + 1, pl.ds(0, NLANE)])
                    x1 = (rows_v[par, lr, pl.ds(NLANE, NLANE)]
                          + rows_v[par, lr + 1, pl.ds(NLANE, NLANE)])
                    x2 = (rows_v[par, lr, pl.ds(2 * NLANE, NLANE)]
                          + rows_v[par, lr + 1, pl.ds(2 * NLANE, NLANE)])
                    x3 = (rows_v[par, lr, pl.ds(3 * NLANE, NLANE)]
                          + rows_v[par, lr + 1, pl.ds(3 * NLANE, NLANE)])
                    return (aa[0] + x0, aa[1] + x1,
                            aa[2] + x2, aa[3] + x3)

                def row_body(r, aa):
                    lr = r - base
                    return (aa[0] + rows_v[par, lr, pl.ds(0, NLANE)],
                            aa[1] + rows_v[par, lr, pl.ds(NLANE, NLANE)],
                            aa[2] + rows_v[par, lr, pl.ds(2 * NLANE, NLANE)],
                            aa[3] + rows_v[par, lr, pl.ds(3 * NLANE, NLANE)])

                a0, a1, a2, a3 = lax.fori_loop(0, npair, row2_body,
                                               (a0, a1, a2, a3))
                a0, a1, a2, a3 = lax.fori_loop(lo + 2 * npair, hi,
                                               row_body,
                                               (a0, a1, a2, a3))
                done = o_hi <= lim
                srow = jnp.where(done, sb, jnp.int32(NB))
                out_v[srow, pl.ds(0, NLANE)] = a0
                out_v[srow, pl.ds(NLANE, NLANE)] = a1
                out_v[srow, pl.ds(2 * NLANE, NLANE)] = a2
                out_v[srow, pl.ds(3 * NLANE, NLANE)] = a3
                zero = jnp.zeros((NLANE,), jnp.float32)
                a0 = jnp.where(done, zero, a0)
                a1 = jnp.where(done, zero, a1)
                a2 = jnp.where(done, zero, a2)
                a3 = jnp.where(done, zero, a3)
                return (a0, a1, a2, a3)

            a0, a1, a2, a3 = lax.fori_loop(s, s_end, bag_body,
                                           (a0, a1, a2, a3))
            last = jnp.maximum(s_end - 1, s)
            incomplete = _sload(offs_v, last + 1) > lim
            s = jnp.where(s_end > s,
                          s_end - incomplete.astype(jnp.int32), s)
            return (s, a0, a1, a2, a3)

        zero = jnp.zeros((NLANE,), jnp.float32)
        lax.fori_loop(0, nsub, sub_loop,
                      (jnp.int32(0), zero, zero, zero, zero))

        pltpu.sync_copy(out_v.at[pl.ds(0, NB)],
                        out_hbm.at[pl.ds(bag0, NB), f])
        return 0

    lax.fori_loop(0, IPW, item_loop, 0)


@jax.jit
def _ebc_sc(vals_pad, offs_pad, tab3):
    mesh = plsc.VectorSubcoreMesh(core_axis_name="c", subcore_axis_name="s")
    return pl.kernel(
        _sc_body,
        out_type=jax.ShapeDtypeStruct((B, F, D), jnp.float32),
        mesh=mesh,
        compiler_params=pltpu.CompilerParams(use_tc_tiling_on_sc=False),
        scratch_types=[
            pltpu.VMEM((NB + 32,), jnp.int32),
            pltpu.VMEM((2 * C,), jnp.int32),
            pltpu.VMEM((2, C, D), jnp.float32),
            pltpu.VMEM((NB + 1, D), jnp.float32),
            pltpu.SemaphoreType.DMA((2,)),
        ],
    )(vals_pad, offs_pad, tab3)


def kernel(values, offsets, tables):
    total = values.shape[0]
    vals_pad = jnp.concatenate(
        [values, jnp.zeros((C + 8,), jnp.int32)])
    offs_pad = jnp.concatenate(
        [offsets, jnp.full((NB + 64,), jnp.int32(total))])
    pooled = _ebc_sc(vals_pad, offs_pad, tables)
    return pooled.reshape(B, F * D)


# speedup vs baseline: 1.3058x; 1.0032x over previous
import jax
import jax.numpy as jnp
from jax import lax
from jax.experimental import pallas as pl
from jax.experimental.pallas import tpu as pltpu
from jax.experimental.pallas import tpu_sc as plsc

F = 26
B = 4096
V = 100000
D = 64
NLANE = 16
NC = 2
NS = 16
NW = NC * NS
NB = 256
IPF = B // NB
NITEMS = F * IPF
IPW = NITEMS // NW
C = 768
CJ = C // 128
KD = D // NLANE


def _sload(ref, i):
    return ref[pl.ds(i, NLANE)][0]


def _sc_body(vals_hbm, offs_hbm, tab_hbm, out_hbm,
             offs_v, idx_v, rows_v, out_v, sems):
    w = lax.axis_index("s") * NC + lax.axis_index("c")

    def item_loop(i, _):
        item = w * IPW + i
        f = item // IPF
        bag0 = (item % IPF) * NB
        goff = f * B + bag0

        pltpu.sync_copy(offs_hbm.at[pl.ds(goff, NB + 32)], offs_v)

        def zero_loop(z, _):
            for k in range(KD):
                out_v[z, pl.ds(k * NLANE, NLANE)] = jnp.zeros((NLANE,),
                                                              jnp.float32)
            return 0

        lax.fori_loop(0, NB, zero_loop, 0)

        p_start = _sload(offs_v, 0)
        p_end = _sload(offs_v, NB)
        p8 = (p_start // 8) * 8
        nsub = (p_end - p8 + (C - 1)) // C

        def fire(fsub, par):
            base = p8 + fsub * C
            pltpu.sync_copy(vals_hbm.at[pl.ds(base, C)],
                            idx_v.at[pl.ds(par * C, C)])
            for j in range(CJ):
                pltpu.async_copy(
                    tab_hbm.at[f].at[idx_v.at[pl.ds(par * C + j * 128,
                                                    128)]],
                    rows_v.at[par, pl.ds(j * 128, 128)], sems.at[par])

        @pl.when(nsub > 0)
        def _():
            fire(jnp.int32(0), jnp.int32(0))

        def sub_loop(sub, carry):
            s, a0, a1, a2, a3 = carry
            par = lax.rem(sub, 2)

            @pl.when(sub + 1 < nsub)
            def _():
                fire(sub + 1, 1 - par)

            pltpu.make_async_copy(
                tab_hbm.at[0].at[pl.ds(0, C)], rows_v.at[par],
                sems.at[par]).wait()

            base = p8 + sub * C
            lim = base + C

            def bs_step(_, lh):
                blo, bhi = lh
                mid = (blo + bhi) // 2
                pred = _sload(offs_v, mid) < lim
                blo = jnp.where(pred, mid + 1, blo)
                bhi = jnp.where(pred, bhi, mid)
                return (blo, bhi)

            s_end, _ = lax.fori_loop(0, 9, bs_step, (s, jnp.int32(NB)))

            def bag_body(sb, a):
                a0, a1, a2, a3 = a
                o_pair = offs_v[pl.ds(sb, NLANE)]
                o_lo = o_pair[0]
                o_hi = o_pair[1]
                lo = jnp.maximum(o_lo, base)
                hi = jnp.minimum(o_hi, lim)

                lob = lo - base
                npair = (hi - lo) // 2

                def row2_body(t, aa):
                    lr = lob + 2 * t
                    x0 = (rows_v[par, lr, pl.ds(0, NLANE)]
                          + rows_v[par, lr + 1, pl.ds(0, NLANE)])
                    x1 = (rows_v[par, lr, pl.ds(NLANE, NLANE)]
                          + rows_v[par, lr + 1, pl.ds(NLANE, NLANE)])
                    x2 = (rows_v[par, lr, pl.ds(2 * NLANE, NLANE)]
                          + rows_v[par, lr + 1, pl.ds(2 * NLANE, NLANE)])
                    x3 = (rows_v[par, lr, pl.ds(3 * NLANE, NLANE)]
                          + rows_v[par, lr + 1, pl.ds(3 * NLANE, NLANE)])
                    return (aa[0] + x0, aa[1] + x1,
                            aa[2] + x2, aa[3] + x3)

                def row_body(r, aa):
                    lr = r - base
                    return (aa[0] + rows_v[par, lr, pl.ds(0, NLANE)],
                            aa[1] + rows_v[par, lr, pl.ds(NLANE, NLANE)],
                            aa[2] + rows_v[par, lr, pl.ds(2 * NLANE, NLANE)],
                            aa[3] + rows_v[par, lr, pl.ds(3 * NLANE, NLANE)])

                a0, a1, a2, a3 = lax.fori_loop(0, npair, row2_body,
                                               (a0, a1, a2, a3))
                a0, a1, a2, a3 = lax.fori_loop(lo + 2 * npair, hi,
                                               row_body,
                                               (a0, a1, a2, a3))
                done = o_hi <= lim
                srow = jnp.where(done, sb, jnp.int32(NB))
                out_v[srow, pl.ds(0, NLANE)] = a0
                out_v[srow, pl.ds(NLANE, NLANE)] = a1
                out_v[srow, pl.ds(2 * NLANE, NLANE)] = a2
                out_v[srow, pl.ds(3 * NLANE, NLANE)] = a3
                zero = jnp.zeros((NLANE,), jnp.float32)
                a0 = jnp.where(done, zero, a0)
                a1 = jnp.where(done, zero, a1)
                a2 = jnp.where(done, zero, a2)
                a3 = jnp.where(done, zero, a3)
                return (a0, a1, a2, a3)

            a0, a1, a2, a3 = lax.fori_loop(s, s_end, bag_body,
                                           (a0, a1, a2, a3))
            last = jnp.maximum(s_end - 1, s)
            incomplete = _sload(offs_v, last + 1) > lim
            s = jnp.where(s_end > s,
                          s_end - incomplete.astype(jnp.int32), s)
            return (s, a0, a1, a2, a3)

        zero = jnp.zeros((NLANE,), jnp.float32)
        lax.fori_loop(0, nsub, sub_loop,
                      (jnp.int32(0), zero, zero, zero, zero))

        pltpu.sync_copy(out_v.at[pl.ds(0, NB)],
                        out_hbm.at[pl.ds(bag0, NB), f])
        return 0

    lax.fori_loop(0, IPW, item_loop, 0)


@jax.jit
def _ebc_sc(vals_pad, offs_pad, tab3):
    mesh = plsc.VectorSubcoreMesh(core_axis_name="c", subcore_axis_name="s")
    return pl.kernel(
        _sc_body,
        out_type=jax.ShapeDtypeStruct((B, F, D), jnp.float32),
        mesh=mesh,
        compiler_params=pltpu.CompilerParams(use_tc_tiling_on_sc=False),
        scratch_types=[
            pltpu.VMEM((NB + 32,), jnp.int32),
            pltpu.VMEM((2 * C,), jnp.int32),
            pltpu.VMEM((2, C, D), jnp.float32),
            pltpu.VMEM((NB + 1, D), jnp.float32),
            pltpu.SemaphoreType.DMA((2,)),
        ],
    )(vals_pad, offs_pad, tab3)


def kernel(values, offsets, tables):
    total = values.shape[0]
    vals_pad = jnp.concatenate(
        [values, jnp.zeros((C + 8,), jnp.int32)])
    offs_pad = jnp.concatenate(
        [offsets, jnp.full((NB + 64,), jnp.int32(total))])
    pooled = _ebc_sc(vals_pad, offs_pad, tables)
    return pooled.reshape(B, F * D)
